# baseline (device time: 20025 ns/iter reference)
import jax
import jax.numpy as jnp
from jax import lax
from jax.experimental import pallas as pl
from jax.experimental.pallas import tpu as pltpu

N_DEV = 8
N_TOK = 1024
D_IN = 256
D_OUT = 512
E_TOTAL = 32
E_LOCAL = E_TOTAL // N_DEV
CAP = 25
TOK_PER = N_TOK // N_DEV


def kernel(x, router_W, route_idx, expert_W):
    del router_W

    def body(x_ref, e_ref, w_ref, out_ref, partial_ref, comm_ref, xcat_ref,
             send_sems, recv_sems):
        my_pos = lax.axis_index("i")

        barrier = pltpu.get_barrier_semaphore()
        for d in range(N_DEV):
            @pl.when(my_pos != d)
            def _():
                pl.semaphore_signal(
                    barrier, inc=1,
                    device_id=(d,), device_id_type=pl.DeviceIdType.MESH,
                )
        pl.semaphore_wait(barrier, N_DEV - 1)

        e = e_ref[...]
        oh = (e == lax.broadcasted_iota(jnp.int32, (N_TOK, E_TOTAL), 1))
        ohb = oh.astype(jnp.bfloat16)
        tri = (lax.broadcasted_iota(jnp.int32, (N_TOK, N_TOK), 0)
               > lax.broadcasted_iota(jnp.int32, (N_TOK, N_TOK), 1))
        pos = jnp.dot(tri.astype(jnp.bfloat16), ohb,
                      preferred_element_type=jnp.float32)
        rank = jnp.sum(pos * oh.astype(jnp.float32), axis=1, keepdims=True)
        lid = jnp.where(
            jnp.logical_and(rank < CAP,
                            (e // E_LOCAL) == my_pos),
            e - my_pos * E_LOCAL, -1)

        xb = x_ref[...].astype(jnp.bfloat16)
        xcat_ref[...] = jnp.concatenate(
            [xb * (lid == ee).astype(jnp.bfloat16) for ee in range(E_LOCAL)],
            axis=1)
        wcat = w_ref[...].astype(jnp.bfloat16).reshape(
            E_LOCAL * D_IN, D_OUT)

        def compute_block(k):
            return jnp.dot(xcat_ref[pl.ds(k * TOK_PER, TOK_PER), :], wcat,
                           preferred_element_type=jnp.float32)

        for j in range(1, N_DEV):
            k = (my_pos + j) % N_DEV
            accb = compute_block(k)
            partial_ref[pl.ds(k * TOK_PER, TOK_PER), :] = accb.astype(
                jnp.bfloat16)
            rdma = pltpu.make_async_remote_copy(
                src_ref=partial_ref.at[pl.ds(k * TOK_PER, TOK_PER), :],
                dst_ref=comm_ref.at[my_pos],
                send_sem=send_sems.at[k],
                recv_sem=recv_sems.at[my_pos],
                device_id=(k,),
                device_id_type=pl.DeviceIdType.MESH,
            )
            rdma.start()

        total = compute_block(my_pos)

        for j in range(1, N_DEV):
            d = (my_pos + j) % N_DEV
            recv = pltpu.make_async_remote_copy(
                src_ref=partial_ref.at[pl.ds(0, TOK_PER), :],
                dst_ref=comm_ref.at[d],
                send_sem=send_sems.at[my_pos],
                recv_sem=recv_sems.at[d],
                device_id=(d,),
                device_id_type=pl.DeviceIdType.MESH,
            )
            recv.wait_recv()
            total += comm_ref[pl.ds(d, 1), :, :].reshape(
                TOK_PER, D_OUT).astype(jnp.float32)
        out_ref[...] = total

        for j in range(1, N_DEV):
            k = (my_pos + j) % N_DEV
            send = pltpu.make_async_remote_copy(
                src_ref=partial_ref.at[pl.ds(k * TOK_PER, TOK_PER), :],
                dst_ref=comm_ref.at[my_pos],
                send_sem=send_sems.at[k],
                recv_sem=recv_sems.at[my_pos],
                device_id=(k,),
                device_id_type=pl.DeviceIdType.MESH,
            )
            send.wait_send()

    return pl.pallas_call(
        body,
        out_shape=jax.ShapeDtypeStruct((TOK_PER, D_OUT), jnp.float32),
        in_specs=[
            pl.BlockSpec(memory_space=pltpu.VMEM),
            pl.BlockSpec(memory_space=pltpu.VMEM),
            pl.BlockSpec(memory_space=pltpu.VMEM),
        ],
        out_specs=pl.BlockSpec(memory_space=pltpu.VMEM),
        scratch_shapes=[
            pltpu.VMEM((N_TOK, D_OUT), jnp.bfloat16),
            pltpu.VMEM((N_DEV, TOK_PER, D_OUT), jnp.bfloat16),
            pltpu.VMEM((N_TOK, E_LOCAL * D_IN), jnp.bfloat16),
            pltpu.SemaphoreType.DMA((N_DEV,)),
            pltpu.SemaphoreType.DMA((N_DEV,)),
        ],
        compiler_params=pltpu.CompilerParams(collective_id=0),
    )(x, route_idx, expert_W)


# device time: 15737 ns/iter; 1.2725x vs baseline; 1.2725x over previous
import jax
import jax.numpy as jnp
from jax import lax
from jax.experimental import pallas as pl
from jax.experimental.pallas import tpu as pltpu

N_DEV = 8
N_TOK = 1024
D_IN = 256
D_OUT = 512
E_TOTAL = 32
E_LOCAL = E_TOTAL // N_DEV
CAP = 25
TOK_PER = N_TOK // N_DEV
FRAME = 48


def kernel(x, router_W, route_idx, expert_W):
    del router_W

    def body(x_ref, e_ref, w_ref, out_ref, xcat_ref, slot_ref, sm_ref,
             sendbuf_ref, recvbuf_ref, send_sems, recv_sems):
        my_pos = lax.axis_index("i")

        barrier = pltpu.get_barrier_semaphore()
        for d in range(N_DEV):
            @pl.when(my_pos != d)
            def _():
                pl.semaphore_signal(
                    barrier, inc=1,
                    device_id=(d,), device_id_type=pl.DeviceIdType.MESH,
                )

        e = e_ref[...]
        oh = (e == lax.broadcasted_iota(jnp.int32, (N_TOK, E_TOTAL), 1))
        ri = lax.broadcasted_iota(jnp.int32, (N_TOK, N_TOK), 0)
        ci = lax.broadcasted_iota(jnp.int32, (N_TOK, N_TOK), 1)
        tri = (ri > ci).astype(jnp.bfloat16)
        pos = jnp.dot(tri, oh.astype(jnp.bfloat16),
                      preferred_element_type=jnp.float32)
        rank = jnp.sum(pos * oh.astype(jnp.float32), axis=1, keepdims=True)
        kept = rank < CAP

        owner = e // E_LOCAL
        dev_iota = lax.broadcasted_iota(jnp.int32, (N_TOK, N_DEV), 1)
        ind_all = jnp.logical_and(kept, owner == dev_iota)
        btri = jnp.logical_and(ri > ci, (ri // TOK_PER) == (ci // TOK_PER))
        pr8 = jnp.dot(btri.astype(jnp.bfloat16), ind_all.astype(jnp.bfloat16),
                      preferred_element_type=jnp.float32)
        slot_ref[...] = jnp.where(ind_all, pr8, -1.0)
        my_col = (dev_iota == my_pos).astype(jnp.float32)
        sm_ref[...] = jnp.sum(
            jnp.where(ind_all, pr8, -1.0) * my_col, axis=1, keepdims=True)

        lid = jnp.where(
            jnp.logical_and(kept, owner == my_pos),
            e - my_pos * E_LOCAL, -1)

        xb = x_ref[...].astype(jnp.bfloat16)
        xcat_ref[...] = jnp.concatenate(
            [xb * (lid == ee).astype(jnp.bfloat16) for ee in range(E_LOCAL)],
            axis=1)
        wcat = w_ref[...].astype(jnp.bfloat16).reshape(
            E_LOCAL * D_IN, D_OUT)

        frame_iota = lax.broadcasted_iota(
            jnp.int32, (TOK_PER, FRAME), 1).astype(jnp.float32)

        def compute_block(k):
            return jnp.dot(xcat_ref[pl.ds(k * TOK_PER, TOK_PER), :], wcat,
                           preferred_element_type=jnp.float32)

        for j in range(1, N_DEV):
            k = (my_pos + j) % N_DEV
            accb = compute_block(k).astype(jnp.bfloat16)
            smk = sm_ref[pl.ds(k * TOK_PER, TOK_PER), :]
            gat = (smk == frame_iota).astype(jnp.bfloat16)
            packed = lax.dot_general(
                gat, accb, ((( 0,), (0,)), ((), ())),
                preferred_element_type=jnp.float32)
            sendbuf_ref[pl.ds(k, 1), :, :] = packed.astype(
                jnp.bfloat16).reshape(1, FRAME, D_OUT)
            if j == 1:
                pl.semaphore_wait(barrier, N_DEV - 1)
            rdma = pltpu.make_async_remote_copy(
                src_ref=sendbuf_ref.at[k],
                dst_ref=recvbuf_ref.at[my_pos],
                send_sem=send_sems.at[k],
                recv_sem=recv_sems.at[my_pos],
                device_id=(k,),
                device_id_type=pl.DeviceIdType.MESH,
            )
            rdma.start()

        total = compute_block(my_pos)
        slot_blk = slot_ref[pl.ds(my_pos * TOK_PER, TOK_PER), :]

        for j in range(1, N_DEV):
            d = (my_pos - j) % N_DEV
            recv = pltpu.make_async_remote_copy(
                src_ref=sendbuf_ref.at[0],
                dst_ref=recvbuf_ref.at[d],
                send_sem=send_sems.at[my_pos],
                recv_sem=recv_sems.at[d],
                device_id=(d,),
                device_id_type=pl.DeviceIdType.MESH,
            )
            recv.wait_recv()
            oh_d = (lax.broadcasted_iota(jnp.int32, (TOK_PER, N_DEV), 1)
                    == d).astype(jnp.float32)
            sd = jnp.sum(slot_blk * oh_d, axis=1, keepdims=True)
            scat = (sd == frame_iota).astype(jnp.bfloat16)
            frame = recvbuf_ref[pl.ds(d, 1), :, :].reshape(FRAME, D_OUT)
            total += jnp.dot(scat, frame, preferred_element_type=jnp.float32)
        out_ref[...] = total

        for j in range(1, N_DEV):
            k = (my_pos + j) % N_DEV
            send = pltpu.make_async_remote_copy(
                src_ref=sendbuf_ref.at[k],
                dst_ref=recvbuf_ref.at[my_pos],
                send_sem=send_sems.at[k],
                recv_sem=recv_sems.at[my_pos],
                device_id=(k,),
                device_id_type=pl.DeviceIdType.MESH,
            )
            send.wait_send()

    return pl.pallas_call(
        body,
        out_shape=jax.ShapeDtypeStruct((TOK_PER, D_OUT), jnp.float32),
        in_specs=[
            pl.BlockSpec(memory_space=pltpu.VMEM),
            pl.BlockSpec(memory_space=pltpu.VMEM),
            pl.BlockSpec(memory_space=pltpu.VMEM),
        ],
        out_specs=pl.BlockSpec(memory_space=pltpu.VMEM),
        scratch_shapes=[
            pltpu.VMEM((N_TOK, E_LOCAL * D_IN), jnp.bfloat16),
            pltpu.VMEM((N_TOK, N_DEV), jnp.float32),
            pltpu.VMEM((N_TOK, 1), jnp.float32),
            pltpu.VMEM((N_DEV, FRAME, D_OUT), jnp.bfloat16),
            pltpu.VMEM((N_DEV, FRAME, D_OUT), jnp.bfloat16),
            pltpu.SemaphoreType.DMA((N_DEV,)),
            pltpu.SemaphoreType.DMA((N_DEV,)),
        ],
        compiler_params=pltpu.CompilerParams(collective_id=0),
    )(x, route_idx, expert_W)
